# baseline (speedup 1.0000x reference)
import functools

import jax
import jax.numpy as jnp
from jax import lax
from jax.experimental import pallas as pl
from jax.experimental.pallas import tpu as pltpu
from jax.experimental.pallas import tpu_sc as plsc

BATCH = 16384
FACTOR = 16
D_MLP = 64


def _sc_info():
    info = plsc.get_sparse_core_info()
    return info.num_cores, info.num_subcores


def _make_sc_gather():
    nc, ns = _sc_info()
    nw = nc * ns
    bpw = BATCH // nw

    mesh = plsc.VectorSubcoreMesh(core_axis_name="c", subcore_axis_name="s")

    @functools.partial(
        pl.kernel,
        mesh=mesh,
        compiler_params=pltpu.CompilerParams(use_tc_tiling_on_sc=False),
        out_type=[
            jax.ShapeDtypeStruct((BATCH, D_MLP), jnp.float32),
            jax.ShapeDtypeStruct((BATCH, D_MLP), jnp.float32),
            jax.ShapeDtypeStruct((BATCH, FACTOR), jnp.float32),
        ],
        scratch_types=[
            pltpu.VMEM((bpw,), jnp.int32),
            pltpu.VMEM((bpw,), jnp.int32),
            pltpu.VMEM((bpw, D_MLP), jnp.float32),
            pltpu.VMEM((bpw, D_MLP), jnp.float32),
            pltpu.VMEM((bpw, FACTOR), jnp.float32),
            pltpu.VMEM((bpw, FACTOR), jnp.float32),
            pltpu.SemaphoreType.DMA,
        ],
    )
    def sc_gather(user_hbm, item_hbm, ug_hbm, ig_hbm, um_hbm, im_hbm,
                  mu_out, mi_out, gmf_out,
                  idx_u, idx_i, mu_v, mi_v, gu_v, gi_v, sem):
        wid = lax.axis_index("s") * nc + lax.axis_index("c")
        base = wid * bpw
        pltpu.sync_copy(user_hbm.at[pl.ds(base, bpw)], idx_u)
        pltpu.sync_copy(item_hbm.at[pl.ds(base, bpw)], idx_i)
        c1 = pltpu.async_copy(um_hbm.at[idx_u], mu_v, sem)
        c2 = pltpu.async_copy(im_hbm.at[idx_i], mi_v, sem)
        c3 = pltpu.async_copy(ug_hbm.at[idx_u], gu_v, sem)
        c4 = pltpu.async_copy(ig_hbm.at[idx_i], gi_v, sem)
        c1.wait()
        c2.wait()
        c3.wait()
        c4.wait()
        pltpu.sync_copy(mu_v, mu_out.at[pl.ds(base, bpw)])
        pltpu.sync_copy(mi_v, mi_out.at[pl.ds(base, bpw)])

        def body(i, _):
            gu_v[i] = gu_v[i] * gi_v[i]
            return 0

        lax.fori_loop(0, bpw, body, 0)
        pltpu.sync_copy(gu_v, gmf_out.at[pl.ds(base, bpw)])

    return sc_gather


_BT = 2048


def _tc_body(mu_ref, mi_ref, gmf_ref, w1a_ref, w1b_ref, b1_ref,
             w2_ref, b2_ref, w3_ref, b3_ref, wnm_ref, wng_ref, bn_ref,
             out_ref):
    hp = jax.lax.Precision.HIGHEST
    h = jnp.dot(mu_ref[...], w1a_ref[...], precision=hp,
                preferred_element_type=jnp.float32)
    h = h + jnp.dot(mi_ref[...], w1b_ref[...], precision=hp,
                    preferred_element_type=jnp.float32)
    h = jnp.maximum(h + b1_ref[...], 0.0)
    h = jnp.maximum(jnp.dot(h, w2_ref[...], precision=hp,
                            preferred_element_type=jnp.float32) + b2_ref[...], 0.0)
    h = jnp.maximum(jnp.dot(h, w3_ref[...], precision=hp,
                            preferred_element_type=jnp.float32) + b3_ref[...], 0.0)
    logit = (jnp.dot(h, wnm_ref[...], precision=hp,
                     preferred_element_type=jnp.float32)
             + jnp.dot(gmf_ref[...], wng_ref[...], precision=hp,
                       preferred_element_type=jnp.float32)
             + bn_ref[...])
    out_ref[...] = jax.nn.sigmoid(logit)


def _tc_mlp(mu, mi, gmf, w1a, w1b, b1, w2t, b2, w3t, b3, wnm, wng, bn):
    grid = (BATCH // _BT,)
    full = lambda shape: pl.BlockSpec(shape, lambda i: (0, 0))
    return pl.pallas_call(
        _tc_body,
        grid=grid,
        in_specs=[
            pl.BlockSpec((_BT, D_MLP), lambda i: (i, 0)),
            pl.BlockSpec((_BT, D_MLP), lambda i: (i, 0)),
            pl.BlockSpec((_BT, FACTOR), lambda i: (i, 0)),
            full((D_MLP, D_MLP)),
            full((D_MLP, D_MLP)),
            full((1, D_MLP)),
            full((D_MLP, 32)),
            full((1, 32)),
            full((32, FACTOR)),
            full((1, FACTOR)),
            full((FACTOR, 1)),
            full((FACTOR, 1)),
            full((1, 1)),
        ],
        out_specs=pl.BlockSpec((_BT, 1), lambda i: (i, 0)),
        out_shape=jax.ShapeDtypeStruct((BATCH, 1), jnp.float32),
    )(mu, mi, gmf, w1a, w1b, b1, w2t, b2, w3t, b3, wnm, wng, bn)


def kernel(user, item, user_embed_GMF, item_embed_GMF, user_embed_MLP,
           item_embed_MLP, W1, b1, W2, b2, W3, b3, Wn, bn):
    user = user.astype(jnp.int32)
    item = item.astype(jnp.int32)
    mu, mi, gmf = _make_sc_gather()(
        user, item, user_embed_GMF, item_embed_GMF,
        user_embed_MLP, item_embed_MLP)
    w1a = W1[:, :D_MLP].T
    w1b = W1[:, D_MLP:].T
    w2t = W2.T
    w3t = W3.T
    wnm = Wn[0, :FACTOR].reshape(FACTOR, 1)
    wng = Wn[0, FACTOR:].reshape(FACTOR, 1)
    return _tc_mlp(mu, mi, gmf,
                   w1a, w1b, b1.reshape(1, -1),
                   w2t, b2.reshape(1, -1),
                   w3t, b3.reshape(1, -1),
                   wnm, wng, bn.reshape(1, 1))
